# baseline (speedup 1.0000x reference)
import functools

import jax
import jax.numpy as jnp
from jax import lax
from jax.experimental import pallas as pl
from jax.experimental.pallas import tpu as pltpu
from jax.experimental.pallas import tpu_sc as plsc

NC, NS = 2, 16
NW = NC * NS
CH = 40
BM = 512


def _proj_body(x_ref, w_ref, b_ref, o_ref):
    o_ref[...] = (
        jnp.dot(
            x_ref[...].astype(jnp.bfloat16),
            w_ref[...],
            preferred_element_type=jnp.float32,
        )
        + b_ref[...]
    )


def _mask_body(m_ref, o_ref):
    o_ref[...] = m_ref[...] == 0


@functools.cache
def _gather_call(total_rows, out_dim):
    n_per_w = total_rows // NW
    n_chunks = n_per_w // CH
    mesh = plsc.VectorSubcoreMesh(core_axis_name="c", subcore_axis_name="s")

    n_pairs = n_chunks // 2

    @functools.partial(
        pl.kernel,
        out_type=jax.ShapeDtypeStruct((total_rows, out_dim), jnp.float32),
        mesh=mesh,
        scratch_types=[
            pltpu.VMEM((n_chunks, CH), jnp.int32),
            pltpu.VMEM((CH, out_dim), jnp.float32),
            pltpu.VMEM((CH, out_dim), jnp.float32),
            pltpu.SemaphoreType.DMA,
            pltpu.SemaphoreType.DMA,
            pltpu.SemaphoreType.DMA,
            pltpu.SemaphoreType.DMA,
        ],
    )
    def gk(tbl_hbm, idx_hbm, out_hbm, idx_v, buf0, buf1, gs0, gs1, os0, os1):
        wid = lax.axis_index("s") * NC + lax.axis_index("c")
        base = wid * n_per_w
        pltpu.sync_copy(idx_hbm.at[wid], idx_v)

        def gather(a, buf, sem):
            return pltpu.make_async_copy(tbl_hbm.at[idx_v.at[a]], buf, sem)

        def writeback(a, buf, sem):
            return pltpu.make_async_copy(
                buf, out_hbm.at[pl.ds(base + a * CH, CH)], sem)

        gather(0, buf0, gs0).start()

        def body(g, carry):
            a = 2 * g
            gather(a, buf0, gs0).wait()
            writeback(a, buf0, os0).start()

            @pl.when(g > 0)
            def _():
                writeback(a - 1, buf1, os1).wait()

            gather(a + 1, buf1, gs1).start()
            gather(a + 1, buf1, gs1).wait()
            writeback(a + 1, buf1, os1).start()

            @pl.when(g + 1 < n_pairs)
            def _():
                writeback(a, buf0, os0).wait()
                gather(a + 2, buf0, gs0).start()

            return carry

        lax.fori_loop(0, n_pairs, body, 0)
        writeback(n_chunks - 2, buf0, os0).wait()
        writeback(n_chunks - 1, buf1, os1).wait()

    return gk


def kernel(ref_expr_inds, attention_mask, emb_table, W, b):
    B, S = ref_expr_inds.shape
    vocab, lang_dim = emb_table.shape
    out_dim = W.shape[1]
    total = B * S

    proj = pl.pallas_call(
        _proj_body,
        grid=(pl.cdiv(vocab, BM),),
        in_specs=[
            pl.BlockSpec((BM, lang_dim), lambda i: (i, 0)),
            pl.BlockSpec((lang_dim, out_dim), lambda i: (0, 0)),
            pl.BlockSpec((1, out_dim), lambda i: (0, 0)),
        ],
        out_specs=pl.BlockSpec((BM, out_dim), lambda i: (i, 0)),
        out_shape=jax.ShapeDtypeStruct((vocab, out_dim), jnp.float32),
    )(emb_table, W.astype(jnp.bfloat16), b.reshape(1, out_dim))

    idx3 = ref_expr_inds.T.reshape(NW, total // NW // CH, CH)
    gathered = _gather_call(total, out_dim)(proj, idx3)
    y = gathered.reshape(S, B, out_dim).transpose(1, 0, 2)

    pad_mask = pl.pallas_call(
        _mask_body,
        out_shape=jax.ShapeDtypeStruct((B, S), jnp.bool_),
    )(attention_mask)
    return (y, pad_mask)


# speedup vs baseline: 1.0609x; 1.0609x over previous
import functools

import jax
import jax.numpy as jnp
from jax import lax
from jax.experimental import pallas as pl
from jax.experimental.pallas import tpu as pltpu
from jax.experimental.pallas import tpu_sc as plsc

NC, NS = 2, 16
NW = NC * NS
CH = 40
BM = 1024


def _proj_body(x_ref, w_ref, b_ref, o_ref):
    o_ref[...] = (
        jnp.dot(x_ref[...], w_ref[...], preferred_element_type=jnp.float32)
        + b_ref[...]
    )


def _mask_body(m_ref, o_ref):
    o_ref[...] = m_ref[...] == 0


@functools.cache
def _gather_call(total_rows, out_dim):
    n_per_w = total_rows // NW
    n_chunks = n_per_w // CH
    mesh = plsc.VectorSubcoreMesh(core_axis_name="c", subcore_axis_name="s")

    n_pairs = n_chunks // 2

    @functools.partial(
        pl.kernel,
        out_type=jax.ShapeDtypeStruct((total_rows, out_dim), jnp.float32),
        mesh=mesh,
        scratch_types=[
            pltpu.VMEM((n_chunks, CH), jnp.int32),
            pltpu.VMEM((CH, out_dim), jnp.float32),
            pltpu.VMEM((CH, out_dim), jnp.float32),
            pltpu.SemaphoreType.DMA,
            pltpu.SemaphoreType.DMA,
            pltpu.SemaphoreType.DMA,
            pltpu.SemaphoreType.DMA,
        ],
    )
    def gk(tbl_hbm, idx_hbm, out_hbm, idx_v, buf0, buf1, gs0, gs1, os0, os1):
        wid = lax.axis_index("s") * NC + lax.axis_index("c")
        base = wid * n_per_w
        pltpu.sync_copy(idx_hbm.at[wid], idx_v)

        def gather(a, buf, sem):
            return pltpu.make_async_copy(tbl_hbm.at[idx_v.at[a]], buf, sem)

        def writeback(a, buf, sem):
            return pltpu.make_async_copy(
                buf, out_hbm.at[pl.ds(base + a * CH, CH)], sem)

        gather(0, buf0, gs0).start()

        def body(g, carry):
            a = 2 * g

            @pl.when(g > 0)
            def _():
                writeback(a - 1, buf1, os1).wait()

            gather(a + 1, buf1, gs1).start()
            gather(a, buf0, gs0).wait()
            writeback(a, buf0, os0).start()

            @pl.when(g + 1 < n_pairs)
            def _():
                writeback(a, buf0, os0).wait()
                gather(a + 2, buf0, gs0).start()

            gather(a + 1, buf1, gs1).wait()
            writeback(a + 1, buf1, os1).start()
            return carry

        lax.fori_loop(0, n_pairs, body, 0)
        writeback(n_chunks - 2, buf0, os0).wait()
        writeback(n_chunks - 1, buf1, os1).wait()

    return gk


def kernel(ref_expr_inds, attention_mask, emb_table, W, b):
    B, S = ref_expr_inds.shape
    vocab, lang_dim = emb_table.shape
    out_dim = W.shape[1]
    total = B * S

    proj = pl.pallas_call(
        _proj_body,
        grid=(pl.cdiv(vocab, BM),),
        in_specs=[
            pl.BlockSpec((BM, lang_dim), lambda i: (i, 0)),
            pl.BlockSpec((lang_dim, out_dim), lambda i: (0, 0)),
            pl.BlockSpec((1, out_dim), lambda i: (0, 0)),
        ],
        out_specs=pl.BlockSpec((BM, out_dim), lambda i: (i, 0)),
        out_shape=jax.ShapeDtypeStruct((vocab, out_dim), jnp.float32),
    )(emb_table, W, b.reshape(1, out_dim))

    idx3 = ref_expr_inds.T.reshape(NW, total // NW // CH, CH)
    gathered = _gather_call(total, out_dim)(proj, idx3)
    y = gathered.reshape(S, B, out_dim).transpose(1, 0, 2)

    pad_mask = pl.pallas_call(
        _mask_body,
        out_shape=jax.ShapeDtypeStruct((B, S), jnp.bool_),
    )(attention_mask)
    return (y, pad_mask)
